# baseline (speedup 1.0000x reference)
import jax
import jax.numpy as jnp
from jax.experimental import pallas as pl
from jax.experimental.pallas import tpu as pltpu

_N = 32768
_D = 128
_H = 128
_B = 16
_EPS = 1e-5
_C = 16384
_NB = _N // _C

_B24 = 24
_T00 = (((0,), (0,)), ((), ()))
_T11 = (((1,), (1,)), ((), ()))


def _attn_readout_kernel(seg_ref, ln_ref, feat_ref, gamma_ref, beta_ref,
                         wu_ref, wv_ref, bv_ref, we_ref,
                         out_ref,
                         sum_s, sq_s, fl_s, m_s, s_s, acc_s, sc_s, sh_s,
                         fv_s, wu_s):
    p = pl.program_id(0)
    i = pl.program_id(1)

    @pl.when((p == 0) & (i == 0))
    def _init():
        sum_s[...] = jnp.zeros_like(sum_s)
        sq_s[...] = jnp.zeros_like(sq_s)
        fl_s[...] = jnp.zeros_like(fl_s)
        m_s[...] = jnp.full_like(m_s, -jnp.inf)
        s_s[...] = jnp.zeros_like(s_s)
        acc_s[...] = jnp.zeros_like(acc_s)

    @pl.when(p == 0)
    def _phase0():
        feat = feat_ref[...]
        sq_s[...] += jnp.sum(feat * feat, axis=0, keepdims=True)
        eye24 = (jax.lax.broadcasted_iota(jnp.int32, (_B24, _B24), 0)
                 == jax.lax.broadcasted_iota(jnp.int32, (_B24, _B24), 1))
        ln_col = jnp.sum(eye24 * ln_ref[...], axis=1, keepdims=True)
        rows = i * _C + jax.lax.broadcasted_iota(jnp.int32, (1, _C), 1)
        col24 = jax.lax.broadcasted_iota(jnp.int32, (_B24, 1), 0)
        G = ((rows == ln_col).astype(jnp.float32)
             + (col24 == _B).astype(jnp.float32))
        out24 = jax.lax.dot(G, feat)
        fl_s[...] += out24[:_B, :]
        sum_s[...] += out24[_B:_B + 1, :]

    @pl.when((p == 1) & (i == 0))
    def _mid():
        mean = sum_s[...] / _N
        var = jnp.maximum(sq_s[...] / _N - mean * mean, 0.0)
        scale = gamma_ref[...] * jax.lax.rsqrt(var + _EPS)
        shift = beta_ref[...] - mean * scale
        sc_s[...] = scale
        sh_s[...] = shift
        eyeD = (jax.lax.broadcasted_iota(jnp.int32, (_D, _D), 0)
                == jax.lax.broadcasted_iota(jnp.int32, (_D, _D), 1))
        diag_scale = eyeD.astype(jnp.float32) * scale
        wu_s[...] = jax.lax.dot(diag_scale, wu_ref[...])
        bias_u = jax.lax.dot(shift, wu_ref[...])
        fb_last = fl_s[...] * scale + shift
        fv_s[...] = (jax.lax.dot(fb_last, wv_ref[...])
                     + bv_ref[...] + bias_u)

    @pl.when(p == 1)
    def _phase1():
        feat = feat_ref[...]
        u = jax.lax.dot(feat, wu_s[...])

        seg_row = seg_ref[...][0]
        maskT = (jax.lax.broadcasted_iota(jnp.int32, (_B, 1), 0)
                 == seg_row)
        maskTf = maskT.astype(jnp.float32)

        vb = jax.lax.dot_general(maskTf, fv_s[...], _T00)
        sg = jax.nn.sigmoid(u + vb)
        e_row = jax.lax.dot_general(we_ref[...], sg, _T11)

        neg = jnp.float32(-jnp.inf)
        bm = jnp.max(jnp.where(maskT, e_row, neg),
                     axis=1, keepdims=True)
        m_old = m_s[...]
        m_new = jnp.maximum(m_old, bm)
        resc = jnp.where(m_old >= m_new, 1.0, jnp.exp(m_old - m_new))
        exT = jnp.exp(jnp.where(maskT, e_row - m_new, neg))
        s_s[...] = s_s[...] * resc + jnp.sum(exT, axis=1, keepdims=True)
        acc_s[...] = acc_s[...] * resc + jax.lax.dot(exT, feat)
        m_s[...] = m_new

        @pl.when(i == _NB - 1)
        def _fin():
            sden = s_s[...]
            valid = sden > 0.0
            inv = jnp.where(valid, 1.0 / sden, 0.0)
            out_ref[...] = (acc_s[...] * inv * sc_s[...]
                            + valid.astype(jnp.float32) * sh_s[...])


def kernel(feat, gamma, beta, W_u, W_v, b_v, W_e, segment_ids, last_nodes):
    seg3 = segment_ids.astype(jnp.int32).reshape(_NB, 1, _C)
    ln = jnp.concatenate([last_nodes.astype(jnp.int32),
                          jnp.full((_B24 - _B,), -1, jnp.int32)]).reshape(1, _B24)
    g = gamma.reshape(1, _D).astype(jnp.float32)
    bt = beta.reshape(1, _D).astype(jnp.float32)
    bv = b_v.reshape(1, _H).astype(jnp.float32)
    we = W_e.reshape(1, _H).astype(jnp.float32)

    const = lambda p, i: (0, 0)
    out = pl.pallas_call(
        _attn_readout_kernel,
        grid=(2, _NB),
        in_specs=[
            pl.BlockSpec((1, 1, _C), lambda p, i: (i, 0, 0)),
            pl.BlockSpec((1, _B24), const),
            pl.BlockSpec((_C, _D), lambda p, i: (i, 0)),
            pl.BlockSpec((1, _D), const),
            pl.BlockSpec((1, _D), const),
            pl.BlockSpec((_D, _H), const),
            pl.BlockSpec((_D, _H), const),
            pl.BlockSpec((1, _H), const),
            pl.BlockSpec((1, _H), const),
        ],
        out_specs=pl.BlockSpec((_B, _D), const),
        out_shape=jax.ShapeDtypeStruct((_B, _D), jnp.float32),
        scratch_shapes=[
            pltpu.VMEM((1, _D), jnp.float32),
            pltpu.VMEM((1, _D), jnp.float32),
            pltpu.VMEM((_B, _D), jnp.float32),
            pltpu.VMEM((_B, 1), jnp.float32),
            pltpu.VMEM((_B, 1), jnp.float32),
            pltpu.VMEM((_B, _D), jnp.float32),
            pltpu.VMEM((1, _D), jnp.float32),
            pltpu.VMEM((1, _D), jnp.float32),
            pltpu.VMEM((_B, _H), jnp.float32),
            pltpu.VMEM((_D, _H), jnp.float32),
        ],
    )(seg3, ln, feat.astype(jnp.float32), g, bt,
      W_u.astype(jnp.float32), W_v.astype(jnp.float32), bv, we)
    return out


# speedup vs baseline: 1.0558x; 1.0558x over previous
import jax
import jax.numpy as jnp
from jax.experimental import pallas as pl
from jax.experimental.pallas import tpu as pltpu

_N = 32768
_D = 128
_H = 128
_B = 16
_EPS = 1e-5
_C = 16384
_NB = _N // _C

_T00 = (((0,), (0,)), ((), ()))
_T11 = (((1,), (1,)), ((), ()))


def _attn_readout_kernel(seg_ref, ln_ref, feat_ref, gamma_ref, beta_ref,
                         wu_ref, wv_ref, bv_ref, we_ref,
                         out_ref,
                         sum_s, sq_s, fl_s, m_s, s_s, acc_s, sc_s, sh_s,
                         fv_s, wu_s):
    p = pl.program_id(0)
    i = pl.program_id(1)

    @pl.when((p == 0) & (i == 0))
    def _init():
        sum_s[...] = jnp.zeros_like(sum_s)
        sq_s[...] = jnp.zeros_like(sq_s)
        fl_s[...] = jnp.zeros_like(fl_s)
        m_s[...] = jnp.full_like(m_s, -jnp.inf)
        s_s[...] = jnp.zeros_like(s_s)
        acc_s[...] = jnp.zeros_like(acc_s)

    @pl.when(p == 0)
    def _phase0():
        feat = feat_ref[...]
        sum_s[...] += jnp.sum(feat, axis=0, keepdims=True)
        sq_s[...] += jnp.sum(feat * feat, axis=0, keepdims=True)
        eyeB = (jax.lax.broadcasted_iota(jnp.int32, (_B, _B), 0)
                == jax.lax.broadcasted_iota(jnp.int32, (_B, _B), 1))
        ln_col = jnp.sum(eyeB * ln_ref[...], axis=1, keepdims=True)
        rows = i * _C + jax.lax.broadcasted_iota(jnp.int32, (1, _C), 1)
        rsT = (rows == ln_col).astype(jnp.float32)
        fl_s[...] += jax.lax.dot(rsT, feat)

    @pl.when((p == 1) & (i == 0))
    def _mid():
        mean = sum_s[...] / _N
        var = jnp.maximum(sq_s[...] / _N - mean * mean, 0.0)
        scale = gamma_ref[...] * jax.lax.rsqrt(var + _EPS)
        shift = beta_ref[...] - mean * scale
        sc_s[...] = scale
        sh_s[...] = shift
        eyeD = (jax.lax.broadcasted_iota(jnp.int32, (_D, _D), 0)
                == jax.lax.broadcasted_iota(jnp.int32, (_D, _D), 1))
        diag_scale = eyeD.astype(jnp.float32) * scale
        wu_s[...] = jax.lax.dot(diag_scale, wu_ref[...])
        bias_u = jax.lax.dot(shift, wu_ref[...])
        fb_last = fl_s[...] * scale + shift
        fv_s[...] = (jax.lax.dot(fb_last, wv_ref[...])
                     + bv_ref[...] + bias_u)

    @pl.when(p == 1)
    def _phase1():
        feat = feat_ref[...]
        u = jax.lax.dot(feat, wu_s[...])

        seg_row = seg_ref[...][0]
        maskT = (jax.lax.broadcasted_iota(jnp.int32, (_B, 1), 0)
                 == seg_row)
        maskTf = maskT.astype(jnp.float32)

        vb = jax.lax.dot_general(maskTf, fv_s[...], _T00)
        sg = jax.nn.sigmoid(u + vb)
        e_row = jax.lax.dot_general(we_ref[...], sg, _T11)

        neg = jnp.float32(-jnp.inf)
        bm = jnp.max(jnp.where(maskT, e_row, neg),
                     axis=1, keepdims=True)
        m_old = m_s[...]
        m_new = jnp.maximum(m_old, bm)
        resc = jnp.where(m_old >= m_new, 1.0, jnp.exp(m_old - m_new))
        exT = jnp.exp(jnp.where(maskT, e_row - m_new, neg))
        s_s[...] = s_s[...] * resc + jnp.sum(exT, axis=1, keepdims=True)
        acc_s[...] = acc_s[...] * resc + jax.lax.dot(exT, feat)
        m_s[...] = m_new

        @pl.when(i == _NB - 1)
        def _fin():
            sden = s_s[...]
            valid = sden > 0.0
            inv = jnp.where(valid, 1.0 / sden, 0.0)
            out_ref[...] = (acc_s[...] * inv * sc_s[...]
                            + valid.astype(jnp.float32) * sh_s[...])


def kernel(feat, gamma, beta, W_u, W_v, b_v, W_e, segment_ids, last_nodes):
    seg3 = segment_ids.astype(jnp.int32).reshape(_NB, 1, _C)
    ln = last_nodes.astype(jnp.int32).reshape(1, _B)
    g = gamma.reshape(1, _D).astype(jnp.float32)
    bt = beta.reshape(1, _D).astype(jnp.float32)
    bv = b_v.reshape(1, _H).astype(jnp.float32)
    we = W_e.reshape(1, _H).astype(jnp.float32)

    const = lambda p, i: (0, 0)
    out = pl.pallas_call(
        _attn_readout_kernel,
        grid=(2, _NB),
        in_specs=[
            pl.BlockSpec((1, 1, _C), lambda p, i: (i, 0, 0)),
            pl.BlockSpec((1, _B), const),
            pl.BlockSpec((_C, _D), lambda p, i: (i, 0)),
            pl.BlockSpec((1, _D), const),
            pl.BlockSpec((1, _D), const),
            pl.BlockSpec((_D, _H), const),
            pl.BlockSpec((_D, _H), const),
            pl.BlockSpec((1, _H), const),
            pl.BlockSpec((1, _H), const),
        ],
        out_specs=pl.BlockSpec((_B, _D), const),
        out_shape=jax.ShapeDtypeStruct((_B, _D), jnp.float32),
        scratch_shapes=[
            pltpu.VMEM((1, _D), jnp.float32),
            pltpu.VMEM((1, _D), jnp.float32),
            pltpu.VMEM((_B, _D), jnp.float32),
            pltpu.VMEM((_B, 1), jnp.float32),
            pltpu.VMEM((_B, 1), jnp.float32),
            pltpu.VMEM((_B, _D), jnp.float32),
            pltpu.VMEM((1, _D), jnp.float32),
            pltpu.VMEM((1, _D), jnp.float32),
            pltpu.VMEM((_B, _H), jnp.float32),
            pltpu.VMEM((_D, _H), jnp.float32),
        ],
    )(seg3, ln, feat.astype(jnp.float32), g, bt,
      W_u.astype(jnp.float32), W_v.astype(jnp.float32), bv, we)
    return out


# speedup vs baseline: 1.0559x; 1.0001x over previous
import jax
import jax.numpy as jnp
from jax.experimental import pallas as pl
from jax.experimental.pallas import tpu as pltpu

_N = 32768
_D = 128
_H = 128
_B = 16
_EPS = 1e-5
_C = 16384
_NB = _N // _C

_T00 = (((0,), (0,)), ((), ()))
_T11 = (((1,), (1,)), ((), ()))


def _attn_readout_kernel(seg_ref, ln_ref, feat_ref, gamma_ref, beta_ref,
                         wu_ref, wv_ref, bv_ref, we_ref,
                         out_ref,
                         sum_s, sq_s, fl_s, m_s, s_s, acc_s, sc_s, sh_s,
                         fv_s, wu_s):
    p = pl.program_id(0)
    i = pl.program_id(1)

    @pl.when((p == 0) & (i == 0))
    def _init():
        sum_s[...] = jnp.zeros_like(sum_s)
        sq_s[...] = jnp.zeros_like(sq_s)
        fl_s[...] = jnp.zeros_like(fl_s)
        m_s[...] = jnp.full_like(m_s, -jnp.inf)
        s_s[...] = jnp.zeros_like(s_s)
        acc_s[...] = jnp.zeros_like(acc_s)

    @pl.when(p == 0)
    def _phase0():
        feat = feat_ref[...]
        f3 = feat.reshape(_C // 8, 8, _D)
        sum_s[...] += jnp.sum(f3, axis=0)
        sq_s[...] += jnp.sum(f3 * f3, axis=0)
        eyeB = (jax.lax.broadcasted_iota(jnp.int32, (_B, _B), 0)
                == jax.lax.broadcasted_iota(jnp.int32, (_B, _B), 1))
        ln_col = jnp.sum(eyeB * ln_ref[...], axis=1, keepdims=True)
        rows = i * _C + jax.lax.broadcasted_iota(jnp.int32, (1, _C), 1)
        rsT = (rows == ln_col).astype(jnp.float32)
        fl_s[...] += jax.lax.dot(rsT, feat)

    @pl.when((p == 1) & (i == 0))
    def _mid():
        mean = jnp.sum(sum_s[...], axis=0, keepdims=True) / _N
        msq = jnp.sum(sq_s[...], axis=0, keepdims=True) / _N
        var = jnp.maximum(msq - mean * mean, 0.0)
        scale = gamma_ref[...] * jax.lax.rsqrt(var + _EPS)
        shift = beta_ref[...] - mean * scale
        sc_s[...] = scale
        sh_s[...] = shift
        eyeD = (jax.lax.broadcasted_iota(jnp.int32, (_D, _D), 0)
                == jax.lax.broadcasted_iota(jnp.int32, (_D, _D), 1))
        diag_scale = eyeD.astype(jnp.float32) * scale
        wu_s[...] = jax.lax.dot(diag_scale, wu_ref[...])
        bias_u = jax.lax.dot(shift, wu_ref[...])
        fb_last = fl_s[...] * scale + shift
        fv_s[...] = (jax.lax.dot(fb_last, wv_ref[...])
                     + bv_ref[...] + bias_u)

    @pl.when(p == 1)
    def _phase1():
        feat = feat_ref[...]
        u = jax.lax.dot(feat, wu_s[...])

        seg_row = seg_ref[...][0]
        maskT = (jax.lax.broadcasted_iota(jnp.int32, (_B, 1), 0)
                 == seg_row)
        maskTf = maskT.astype(jnp.float32)

        vb = jax.lax.dot_general(maskTf, fv_s[...], _T00)
        sg = jax.nn.sigmoid(u + vb)
        e_row = jax.lax.dot_general(we_ref[...], sg, _T11)

        neg = jnp.float32(-jnp.inf)
        bm = jnp.max(jnp.where(maskT, e_row, neg),
                     axis=1, keepdims=True)
        m_old = m_s[...]
        m_new = jnp.maximum(m_old, bm)
        resc = jnp.where(m_old >= m_new, 1.0, jnp.exp(m_old - m_new))
        exT = jnp.exp(jnp.where(maskT, e_row - m_new, neg))
        s_s[...] = s_s[...] * resc + jnp.sum(exT, axis=1, keepdims=True)
        acc_s[...] = acc_s[...] * resc + jax.lax.dot(exT, feat)
        m_s[...] = m_new

        @pl.when(i == _NB - 1)
        def _fin():
            sden = s_s[...]
            valid = sden > 0.0
            inv = jnp.where(valid, 1.0 / sden, 0.0)
            out_ref[...] = (acc_s[...] * inv * sc_s[...]
                            + valid.astype(jnp.float32) * sh_s[...])


def kernel(feat, gamma, beta, W_u, W_v, b_v, W_e, segment_ids, last_nodes):
    seg3 = segment_ids.astype(jnp.int32).reshape(_NB, 1, _C)
    ln = last_nodes.astype(jnp.int32).reshape(1, _B)
    g = gamma.reshape(1, _D).astype(jnp.float32)
    bt = beta.reshape(1, _D).astype(jnp.float32)
    bv = b_v.reshape(1, _H).astype(jnp.float32)
    we = W_e.reshape(1, _H).astype(jnp.float32)

    const = lambda p, i: (0, 0)
    out = pl.pallas_call(
        _attn_readout_kernel,
        grid=(2, _NB),
        in_specs=[
            pl.BlockSpec((1, 1, _C), lambda p, i: (i, 0, 0)),
            pl.BlockSpec((1, _B), const),
            pl.BlockSpec((_C, _D), lambda p, i: (i, 0)),
            pl.BlockSpec((1, _D), const),
            pl.BlockSpec((1, _D), const),
            pl.BlockSpec((_D, _H), const),
            pl.BlockSpec((_D, _H), const),
            pl.BlockSpec((1, _H), const),
            pl.BlockSpec((1, _H), const),
        ],
        out_specs=pl.BlockSpec((_B, _D), const),
        out_shape=jax.ShapeDtypeStruct((_B, _D), jnp.float32),
        scratch_shapes=[
            pltpu.VMEM((8, _D), jnp.float32),
            pltpu.VMEM((8, _D), jnp.float32),
            pltpu.VMEM((_B, _D), jnp.float32),
            pltpu.VMEM((_B, 1), jnp.float32),
            pltpu.VMEM((_B, 1), jnp.float32),
            pltpu.VMEM((_B, _D), jnp.float32),
            pltpu.VMEM((1, _D), jnp.float32),
            pltpu.VMEM((1, _D), jnp.float32),
            pltpu.VMEM((_B, _H), jnp.float32),
            pltpu.VMEM((_D, _H), jnp.float32),
        ],
    )(seg3, ln, feat.astype(jnp.float32), g, bt,
      W_u.astype(jnp.float32), W_v.astype(jnp.float32), bv, we)
    return out
